# Initial kernel scaffold; baseline (speedup 1.0000x reference)
import jax
import jax.numpy as jnp
from jax.experimental import pallas as pl


def kernel(x, edge_index, W1, b1, W_mu, b_mu, W_ls, b_ls):
    raise NotImplementedError("write your pallas kernel here")


# speedup vs baseline: 6.7698x; 6.7698x over previous
import functools

import jax
import jax.numpy as jnp
from jax import lax
from jax.experimental import pallas as pl
from jax.experimental.pallas import tpu as pltpu
from jax.experimental.pallas import tpu_sc as plsc

N = 10000
E = 320000
D = 128
DOUT = 64
NC, NS = 1, 16
NW = NC * NS
LANE = 128
RW = 160
E_PAD = NW * RW * LANE
ROWS = E_PAD // LANE
TRASH = N
NACC = N + 8
RPA = 624
TAIL = N - NS * RPA
ZR = 104

_mesh = plsc.VectorSubcoreMesh(
    core_axis_name="c", subcore_axis_name="s", num_cores=1, num_subcores=NS
)


@functools.partial(
    pl.kernel,
    out_type=jax.ShapeDtypeStruct((N, D), jnp.float32),
    mesh=_mesh,
    scratch_types=[
        pltpu.VMEM((LANE,), jnp.int32),
        pltpu.VMEM((LANE, D), jnp.float32),
        pltpu.VMEM_SHARED((NACC, D), jnp.float32),
    ],
)
def _deg_kernel(dst_hbm, z_hbm, ones_hbm, out_hbm, dstv, ones, acc):
    s = lax.axis_index("s")
    w = s

    pltpu.sync_copy(ones_hbm, ones)

    pltpu.sync_copy(z_hbm.at[pl.ds(s * RPA, RPA)], acc.at[pl.ds(s * RPA, RPA)])

    @pl.when(s == NS - 1)
    def _():
        pltpu.sync_copy(
            z_hbm.at[pl.ds(NS * RPA, TAIL)], acc.at[pl.ds(NS * RPA, TAIL)]
        )

    plsc.subcore_barrier()

    @pl.loop(0, RW)
    def _(j):
        pltpu.sync_copy(dst_hbm.at[pl.ds((w * RW + j) * LANE, LANE)], dstv)
        pltpu.sync_copy(ones, acc.at[dstv], add=True)

    plsc.subcore_barrier()
    pltpu.sync_copy(
        acc.at[pl.ds(s * RPA, RPA)], out_hbm.at[pl.ds(s * RPA, RPA)]
    )

    @pl.when(s == NS - 1)
    def _():
        pltpu.sync_copy(
            acc.at[pl.ds(NS * RPA, TAIL)],
            out_hbm.at[pl.ds(NS * RPA, TAIL)],
        )


@functools.partial(
    pl.kernel,
    out_type=jax.ShapeDtypeStruct((N, D), jnp.float32),
    mesh=_mesh,
    scratch_types=[
        pltpu.VMEM((LANE,), jnp.int32),
        pltpu.VMEM((LANE,), jnp.int32),
        pltpu.VMEM((LANE, D), jnp.float32),
        pltpu.VMEM_SHARED((NACC, D), jnp.float32),
    ],
)
def _prop_kernel(h_hbm, src_hbm, dst_hbm, z_hbm, out_hbm, srcv, dstv, buf, acc):
    s = lax.axis_index("s")
    w = s

    pltpu.sync_copy(z_hbm.at[pl.ds(s * RPA, RPA)], acc.at[pl.ds(s * RPA, RPA)])

    @pl.when(s == NS - 1)
    def _():
        pltpu.sync_copy(
            z_hbm.at[pl.ds(NS * RPA, TAIL)], acc.at[pl.ds(NS * RPA, TAIL)]
        )

    plsc.subcore_barrier()

    @pl.loop(0, RW)
    def _(j):
        pltpu.sync_copy(src_hbm.at[pl.ds((w * RW + j) * LANE, LANE)], srcv)
        pltpu.sync_copy(dst_hbm.at[pl.ds((w * RW + j) * LANE, LANE)], dstv)
        pltpu.sync_copy(h_hbm.at[srcv], buf)
        pltpu.sync_copy(buf, acc.at[dstv], add=True)

    plsc.subcore_barrier()
    pltpu.sync_copy(
        acc.at[pl.ds(s * RPA, RPA)], out_hbm.at[pl.ds(s * RPA, RPA)]
    )

    @pl.when(s == NS - 1)
    def _():
        pltpu.sync_copy(
            acc.at[pl.ds(NS * RPA, TAIL)],
            out_hbm.at[pl.ds(NS * RPA, TAIL)],
        )


BLK = 1000
GRID = N // BLK


def _layer1_body(x_ref, w_ref, pd0_ref, h0s_ref, dis_ref):
    deg = 1.0 + pd0_ref[:, 0:1]
    dis = lax.rsqrt(deg)
    h0 = jnp.dot(
        x_ref[...], w_ref[...], preferred_element_type=jnp.float32,
        precision=lax.Precision.HIGHEST,
    )
    h0s_ref[...] = h0 * dis
    dis_ref[...] = jnp.broadcast_to(dis, (BLK, 16))


_layer1 = pl.pallas_call(
    _layer1_body,
    grid=(GRID,),
    in_specs=[
        pl.BlockSpec((BLK, D), lambda i: (i, 0)),
        pl.BlockSpec((D, D), lambda i: (0, 0)),
        pl.BlockSpec((BLK, D), lambda i: (i, 0)),
    ],
    out_specs=[
        pl.BlockSpec((BLK, D), lambda i: (i, 0)),
        pl.BlockSpec((BLK, 16), lambda i: (i, 0)),
    ],
    out_shape=[
        jax.ShapeDtypeStruct((N, D), jnp.float32),
        jax.ShapeDtypeStruct((N, 16), jnp.float32),
    ],
)


def _layer2_body(p0_ref, h0s_ref, dis_ref, b1_ref, w_ref, h2s_ref):
    disb = dis_ref[:, 0:1]
    h = jnp.maximum(
        (p0_ref[...] + h0s_ref[...]) * disb + b1_ref[...], 0.0
    )
    h2s_ref[...] = jnp.dot(
        h, w_ref[...], preferred_element_type=jnp.float32,
        precision=lax.Precision.HIGHEST,
    ) * disb


_layer2 = pl.pallas_call(
    _layer2_body,
    grid=(GRID,),
    in_specs=[
        pl.BlockSpec((BLK, D), lambda i: (i, 0)),
        pl.BlockSpec((BLK, D), lambda i: (i, 0)),
        pl.BlockSpec((BLK, 16), lambda i: (i, 0)),
        pl.BlockSpec((1, D), lambda i: (0, 0)),
        pl.BlockSpec((D, D), lambda i: (0, 0)),
    ],
    out_specs=pl.BlockSpec((BLK, D), lambda i: (i, 0)),
    out_shape=jax.ShapeDtypeStruct((N, D), jnp.float32),
)


def _final_body(q0_ref, h2s_ref, dis_ref, bmu_ref, bls_ref, mu_ref, ls_ref):
    disb = dis_ref[:, 0:1]
    out = (q0_ref[...] + h2s_ref[...]) * disb
    mu_ref[...] = out[:, :DOUT] + bmu_ref[...]
    ls_ref[...] = out[:, DOUT:] + bls_ref[...]


_final = pl.pallas_call(
    _final_body,
    grid=(GRID,),
    in_specs=[
        pl.BlockSpec((BLK, D), lambda i: (i, 0)),
        pl.BlockSpec((BLK, D), lambda i: (i, 0)),
        pl.BlockSpec((BLK, 16), lambda i: (i, 0)),
        pl.BlockSpec((1, DOUT), lambda i: (0, 0)),
        pl.BlockSpec((1, DOUT), lambda i: (0, 0)),
    ],
    out_specs=[
        pl.BlockSpec((BLK, DOUT), lambda i: (i, 0)),
        pl.BlockSpec((BLK, DOUT), lambda i: (i, 0)),
    ],
    out_shape=[
        jax.ShapeDtypeStruct((N, DOUT), jnp.float32),
        jax.ShapeDtypeStruct((N, DOUT), jnp.float32),
    ],
)


def kernel(x, edge_index, W1, b1, W_mu, b_mu, W_ls, b_ls):
    pad = E_PAD - E
    src_p = jnp.concatenate([edge_index[0], jnp.zeros((pad,), jnp.int32)])
    dst_p = jnp.concatenate([edge_index[1], jnp.full((pad,), TRASH, jnp.int32)])
    wcat = jnp.concatenate([W_mu, W_ls], axis=1)

    zerosD = jnp.zeros((N, D), jnp.float32)
    pd = _deg_kernel(dst_p, zerosD, jnp.ones((LANE, D), jnp.float32))
    h0s, dis = _layer1(x, W1, pd)
    p = _prop_kernel(h0s, src_p, dst_p, zerosD)
    h2s = _layer2(p, h0s, dis, b1.reshape(1, D), wcat)
    q = _prop_kernel(h2s, src_p, dst_p, zerosD)
    mu, ls = _final(q, h2s, dis, b_mu.reshape(1, DOUT), b_ls.reshape(1, DOUT))
    return (mu, ls)


# speedup vs baseline: 8.7218x; 1.2883x over previous
import functools

import jax
import jax.numpy as jnp
from jax import lax
from jax.experimental import pallas as pl
from jax.experimental.pallas import tpu as pltpu
from jax.experimental.pallas import tpu_sc as plsc

N = 10000
E = 320000
D = 128
DOUT = 64
NC, NS = 1, 16
NW = NC * NS
LANE = 128
RW = 160
E_PAD = NW * RW * LANE
ROWS = E_PAD // LANE
TRASH = N
NACC = N + 8
RPA = 624
TAIL = N - NS * RPA
ZR = 104

_mesh = plsc.VectorSubcoreMesh(
    core_axis_name="c", subcore_axis_name="s", num_cores=1, num_subcores=NS
)


@functools.partial(
    pl.kernel,
    out_type=jax.ShapeDtypeStruct((N, D), jnp.float32),
    mesh=_mesh,
    scratch_types=[
        pltpu.VMEM((LANE,), jnp.int32),
        pltpu.VMEM((LANE, D), jnp.float32),
        pltpu.VMEM_SHARED((NACC, D), jnp.float32),
    ],
)
def _deg_kernel(dst_hbm, z_hbm, ones_hbm, out_hbm, dstv, ones, acc):
    s = lax.axis_index("s")
    w = s

    pltpu.sync_copy(ones_hbm, ones)

    pltpu.sync_copy(z_hbm.at[pl.ds(s * RPA, RPA)], acc.at[pl.ds(s * RPA, RPA)])

    @pl.when(s == NS - 1)
    def _():
        pltpu.sync_copy(
            z_hbm.at[pl.ds(NS * RPA, TAIL)], acc.at[pl.ds(NS * RPA, TAIL)]
        )

    plsc.subcore_barrier()

    @pl.loop(0, RW)
    def _(j):
        pltpu.sync_copy(dst_hbm.at[pl.ds((w * RW + j) * LANE, LANE)], dstv)
        pltpu.sync_copy(ones, acc.at[dstv], add=True)

    plsc.subcore_barrier()
    pltpu.sync_copy(
        acc.at[pl.ds(s * RPA, RPA)], out_hbm.at[pl.ds(s * RPA, RPA)]
    )

    @pl.when(s == NS - 1)
    def _():
        pltpu.sync_copy(
            acc.at[pl.ds(NS * RPA, TAIL)],
            out_hbm.at[pl.ds(NS * RPA, TAIL)],
        )


@functools.partial(
    pl.kernel,
    out_type=jax.ShapeDtypeStruct((N, D), jnp.float32),
    mesh=_mesh,
    scratch_types=[
        pltpu.VMEM((LANE,), jnp.int32),
        pltpu.VMEM((LANE,), jnp.int32),
        pltpu.VMEM((LANE,), jnp.int32),
        pltpu.VMEM((LANE,), jnp.int32),
        pltpu.VMEM((LANE, D), jnp.float32),
        pltpu.VMEM((LANE, D), jnp.float32),
        pltpu.VMEM_SHARED((NACC, D), jnp.float32),
        pltpu.SemaphoreType.DMA,
        pltpu.SemaphoreType.DMA,
    ],
)
def _prop_kernel(h_hbm, src_hbm, dst_hbm, z_hbm, out_hbm,
                 srcv0, dstv0, srcv1, dstv1, buf0, buf1, acc, sg0, sg1):
    s = lax.axis_index("s")
    w = s

    pltpu.sync_copy(z_hbm.at[pl.ds(s * RPA, RPA)], acc.at[pl.ds(s * RPA, RPA)])

    @pl.when(s == NS - 1)
    def _():
        pltpu.sync_copy(
            z_hbm.at[pl.ds(NS * RPA, TAIL)], acc.at[pl.ds(NS * RPA, TAIL)]
        )

    plsc.subcore_barrier()

    base = w * RW
    pltpu.sync_copy(src_hbm.at[pl.ds(base * LANE, LANE)], srcv0)
    pltpu.sync_copy(dst_hbm.at[pl.ds(base * LANE, LANE)], dstv0)
    pltpu.async_copy(h_hbm.at[srcv0], buf0, sg0)
    pltpu.sync_copy(src_hbm.at[pl.ds((base + 1) * LANE, LANE)], srcv1)
    pltpu.sync_copy(dst_hbm.at[pl.ds((base + 1) * LANE, LANE)], dstv1)

    @pl.loop(0, RW, step=2)
    def _(j):
        pltpu.make_async_copy(h_hbm.at[srcv0], buf0, sg0).wait()
        pltpu.async_copy(h_hbm.at[srcv1], buf1, sg1)
        pltpu.sync_copy(buf0, acc.at[dstv0], add=True)

        @pl.when(j + 2 < RW)
        def _():
            pltpu.sync_copy(
                src_hbm.at[pl.ds((base + j + 2) * LANE, LANE)], srcv0
            )
            pltpu.sync_copy(
                dst_hbm.at[pl.ds((base + j + 2) * LANE, LANE)], dstv0
            )

        pltpu.make_async_copy(h_hbm.at[srcv1], buf1, sg1).wait()

        @pl.when(j + 2 < RW)
        def _():
            pltpu.async_copy(h_hbm.at[srcv0], buf0, sg0)

        pltpu.sync_copy(buf1, acc.at[dstv1], add=True)

        @pl.when(j + 3 < RW)
        def _():
            pltpu.sync_copy(
                src_hbm.at[pl.ds((base + j + 3) * LANE, LANE)], srcv1
            )
            pltpu.sync_copy(
                dst_hbm.at[pl.ds((base + j + 3) * LANE, LANE)], dstv1
            )

    plsc.subcore_barrier()
    pltpu.sync_copy(
        acc.at[pl.ds(s * RPA, RPA)], out_hbm.at[pl.ds(s * RPA, RPA)]
    )

    @pl.when(s == NS - 1)
    def _():
        pltpu.sync_copy(
            acc.at[pl.ds(NS * RPA, TAIL)],
            out_hbm.at[pl.ds(NS * RPA, TAIL)],
        )


BLK = 1000
GRID = N // BLK


def _layer1_body(x_ref, w_ref, pd0_ref, h0s_ref, dis_ref):
    deg = 1.0 + pd0_ref[:, 0:1]
    dis = lax.rsqrt(deg)
    h0 = jnp.dot(
        x_ref[...], w_ref[...], preferred_element_type=jnp.float32,
        precision=lax.Precision.HIGHEST,
    )
    h0s_ref[...] = h0 * dis
    dis_ref[...] = jnp.broadcast_to(dis, (BLK, 16))


_layer1 = pl.pallas_call(
    _layer1_body,
    grid=(GRID,),
    in_specs=[
        pl.BlockSpec((BLK, D), lambda i: (i, 0)),
        pl.BlockSpec((D, D), lambda i: (0, 0)),
        pl.BlockSpec((BLK, D), lambda i: (i, 0)),
    ],
    out_specs=[
        pl.BlockSpec((BLK, D), lambda i: (i, 0)),
        pl.BlockSpec((BLK, 16), lambda i: (i, 0)),
    ],
    out_shape=[
        jax.ShapeDtypeStruct((N, D), jnp.float32),
        jax.ShapeDtypeStruct((N, 16), jnp.float32),
    ],
)


def _layer2_body(p0_ref, h0s_ref, dis_ref, b1_ref, w_ref, h2s_ref):
    disb = dis_ref[:, 0:1]
    h = jnp.maximum(
        (p0_ref[...] + h0s_ref[...]) * disb + b1_ref[...], 0.0
    )
    h2s_ref[...] = jnp.dot(
        h, w_ref[...], preferred_element_type=jnp.float32,
        precision=lax.Precision.HIGHEST,
    ) * disb


_layer2 = pl.pallas_call(
    _layer2_body,
    grid=(GRID,),
    in_specs=[
        pl.BlockSpec((BLK, D), lambda i: (i, 0)),
        pl.BlockSpec((BLK, D), lambda i: (i, 0)),
        pl.BlockSpec((BLK, 16), lambda i: (i, 0)),
        pl.BlockSpec((1, D), lambda i: (0, 0)),
        pl.BlockSpec((D, D), lambda i: (0, 0)),
    ],
    out_specs=pl.BlockSpec((BLK, D), lambda i: (i, 0)),
    out_shape=jax.ShapeDtypeStruct((N, D), jnp.float32),
)


def _final_body(q0_ref, h2s_ref, dis_ref, bmu_ref, bls_ref, mu_ref, ls_ref):
    disb = dis_ref[:, 0:1]
    out = (q0_ref[...] + h2s_ref[...]) * disb
    mu_ref[...] = out[:, :DOUT] + bmu_ref[...]
    ls_ref[...] = out[:, DOUT:] + bls_ref[...]


_final = pl.pallas_call(
    _final_body,
    grid=(GRID,),
    in_specs=[
        pl.BlockSpec((BLK, D), lambda i: (i, 0)),
        pl.BlockSpec((BLK, D), lambda i: (i, 0)),
        pl.BlockSpec((BLK, 16), lambda i: (i, 0)),
        pl.BlockSpec((1, DOUT), lambda i: (0, 0)),
        pl.BlockSpec((1, DOUT), lambda i: (0, 0)),
    ],
    out_specs=[
        pl.BlockSpec((BLK, DOUT), lambda i: (i, 0)),
        pl.BlockSpec((BLK, DOUT), lambda i: (i, 0)),
    ],
    out_shape=[
        jax.ShapeDtypeStruct((N, DOUT), jnp.float32),
        jax.ShapeDtypeStruct((N, DOUT), jnp.float32),
    ],
)


def kernel(x, edge_index, W1, b1, W_mu, b_mu, W_ls, b_ls):
    pad = E_PAD - E
    src_p = jnp.concatenate([edge_index[0], jnp.zeros((pad,), jnp.int32)])
    dst_p = jnp.concatenate([edge_index[1], jnp.full((pad,), TRASH, jnp.int32)])
    wcat = jnp.concatenate([W_mu, W_ls], axis=1)

    zerosD = jnp.zeros((N, D), jnp.float32)
    pd = _deg_kernel(dst_p, zerosD, jnp.ones((LANE, D), jnp.float32))
    h0s, dis = _layer1(x, W1, pd)
    p = _prop_kernel(h0s, src_p, dst_p, zerosD)
    h2s = _layer2(p, h0s, dis, b1.reshape(1, D), wcat)
    q = _prop_kernel(h2s, src_p, dst_p, zerosD)
    mu, ls = _final(q, h2s, dis, b_mu.reshape(1, DOUT), b_ls.reshape(1, DOUT))
    return (mu, ls)


# speedup vs baseline: 10.0726x; 1.1549x over previous
import functools

import jax
import jax.numpy as jnp
from jax import lax
from jax.experimental import pallas as pl
from jax.experimental.pallas import tpu as pltpu
from jax.experimental.pallas import tpu_sc as plsc

N = 10000
E = 320000
D = 128
DOUT = 64
NC, NS = 2, 16
NW = NC * NS
LANE = 128
RW = 80
E_PAD = NW * RW * LANE
ROWS = E_PAD // LANE
TRASH = N
NACC = N + 8
RPA = 624
TAIL = N - NS * RPA
ZR = 104

_mesh = plsc.VectorSubcoreMesh(
    core_axis_name="c", subcore_axis_name="s", num_cores=NC, num_subcores=NS
)


@functools.partial(
    pl.kernel,
    out_type=jax.ShapeDtypeStruct((2 * N, D), jnp.float32),
    mesh=_mesh,
    scratch_types=[
        pltpu.VMEM((LANE,), jnp.int32),
        pltpu.VMEM((LANE, D), jnp.float32),
        pltpu.VMEM_SHARED((NACC, D), jnp.float32),
    ],
)
def _deg_kernel(dst_hbm, z_hbm, ones_hbm, out_hbm, dstv, ones, acc):
    c = lax.axis_index("c")
    s = lax.axis_index("s")
    w = c * NS + s

    pltpu.sync_copy(ones_hbm, ones)

    pltpu.sync_copy(z_hbm.at[pl.ds(s * RPA, RPA)], acc.at[pl.ds(s * RPA, RPA)])

    @pl.when(s == NS - 1)
    def _():
        pltpu.sync_copy(
            z_hbm.at[pl.ds(NS * RPA, TAIL)], acc.at[pl.ds(NS * RPA, TAIL)]
        )

    plsc.subcore_barrier()

    @pl.loop(0, RW)
    def _(j):
        pltpu.sync_copy(dst_hbm.at[pl.ds((w * RW + j) * LANE, LANE)], dstv)
        pltpu.sync_copy(ones, acc.at[dstv], add=True)

    plsc.subcore_barrier()
    pltpu.sync_copy(
        acc.at[pl.ds(s * RPA, RPA)], out_hbm.at[pl.ds(c * N + s * RPA, RPA)]
    )

    @pl.when(s == NS - 1)
    def _():
        pltpu.sync_copy(
            acc.at[pl.ds(NS * RPA, TAIL)],
            out_hbm.at[pl.ds(c * N + NS * RPA, TAIL)],
        )


@functools.partial(
    pl.kernel,
    out_type=jax.ShapeDtypeStruct((2 * N, D), jnp.float32),
    mesh=_mesh,
    scratch_types=[
        pltpu.VMEM((LANE,), jnp.int32),
        pltpu.VMEM((LANE,), jnp.int32),
        pltpu.VMEM((LANE,), jnp.int32),
        pltpu.VMEM((LANE,), jnp.int32),
        pltpu.VMEM((LANE, D), jnp.float32),
        pltpu.VMEM((LANE, D), jnp.float32),
        pltpu.VMEM_SHARED((NACC, D), jnp.float32),
        pltpu.SemaphoreType.DMA,
        pltpu.SemaphoreType.DMA,
    ],
)
def _prop_kernel(h_hbm, src_hbm, dst_hbm, z_hbm, out_hbm,
                 srcv0, dstv0, srcv1, dstv1, buf0, buf1, acc, sg0, sg1):
    c = lax.axis_index("c")
    s = lax.axis_index("s")
    w = c * NS + s

    pltpu.sync_copy(z_hbm.at[pl.ds(s * RPA, RPA)], acc.at[pl.ds(s * RPA, RPA)])

    @pl.when(s == NS - 1)
    def _():
        pltpu.sync_copy(
            z_hbm.at[pl.ds(NS * RPA, TAIL)], acc.at[pl.ds(NS * RPA, TAIL)]
        )

    plsc.subcore_barrier()

    base = w * RW
    pltpu.sync_copy(src_hbm.at[pl.ds(base * LANE, LANE)], srcv0)
    pltpu.sync_copy(dst_hbm.at[pl.ds(base * LANE, LANE)], dstv0)
    pltpu.async_copy(h_hbm.at[srcv0], buf0, sg0)
    pltpu.sync_copy(src_hbm.at[pl.ds((base + 1) * LANE, LANE)], srcv1)
    pltpu.sync_copy(dst_hbm.at[pl.ds((base + 1) * LANE, LANE)], dstv1)

    @pl.loop(0, RW, step=2)
    def _(j):
        pltpu.make_async_copy(h_hbm.at[srcv0], buf0, sg0).wait()
        pltpu.async_copy(h_hbm.at[srcv1], buf1, sg1)
        pltpu.sync_copy(buf0, acc.at[dstv0], add=True)

        @pl.when(j + 2 < RW)
        def _():
            pltpu.sync_copy(
                src_hbm.at[pl.ds((base + j + 2) * LANE, LANE)], srcv0
            )
            pltpu.sync_copy(
                dst_hbm.at[pl.ds((base + j + 2) * LANE, LANE)], dstv0
            )

        pltpu.make_async_copy(h_hbm.at[srcv1], buf1, sg1).wait()

        @pl.when(j + 2 < RW)
        def _():
            pltpu.async_copy(h_hbm.at[srcv0], buf0, sg0)

        pltpu.sync_copy(buf1, acc.at[dstv1], add=True)

        @pl.when(j + 3 < RW)
        def _():
            pltpu.sync_copy(
                src_hbm.at[pl.ds((base + j + 3) * LANE, LANE)], srcv1
            )
            pltpu.sync_copy(
                dst_hbm.at[pl.ds((base + j + 3) * LANE, LANE)], dstv1
            )

    plsc.subcore_barrier()
    pltpu.sync_copy(
        acc.at[pl.ds(s * RPA, RPA)], out_hbm.at[pl.ds(c * N + s * RPA, RPA)]
    )

    @pl.when(s == NS - 1)
    def _():
        pltpu.sync_copy(
            acc.at[pl.ds(NS * RPA, TAIL)],
            out_hbm.at[pl.ds(c * N + NS * RPA, TAIL)],
        )


BLK = 1000
GRID = N // BLK


def _layer1_body(x_ref, w_ref, pd0_ref, pd1_ref, h0s_ref, dis_ref):
    deg = 1.0 + pd0_ref[:, 0:1] + pd1_ref[:, 0:1]
    dis = lax.rsqrt(deg)
    h0 = jnp.dot(
        x_ref[...], w_ref[...], preferred_element_type=jnp.float32,
        precision=lax.Precision.HIGHEST,
    )
    h0s_ref[...] = h0 * dis
    dis_ref[...] = jnp.broadcast_to(dis, (BLK, 16))


_layer1 = pl.pallas_call(
    _layer1_body,
    grid=(GRID,),
    in_specs=[
        pl.BlockSpec((BLK, D), lambda i: (i, 0)),
        pl.BlockSpec((D, D), lambda i: (0, 0)),
        pl.BlockSpec((BLK, D), lambda i: (i, 0)),
        pl.BlockSpec((BLK, D), lambda i: (i + GRID, 0)),
    ],
    out_specs=[
        pl.BlockSpec((BLK, D), lambda i: (i, 0)),
        pl.BlockSpec((BLK, 16), lambda i: (i, 0)),
    ],
    out_shape=[
        jax.ShapeDtypeStruct((N, D), jnp.float32),
        jax.ShapeDtypeStruct((N, 16), jnp.float32),
    ],
)


def _layer2_body(p0_ref, p1_ref, h0s_ref, dis_ref, b1_ref, w_ref, h2s_ref):
    disb = dis_ref[:, 0:1]
    h = jnp.maximum(
        (p0_ref[...] + p1_ref[...] + h0s_ref[...]) * disb + b1_ref[...], 0.0
    )
    h2s_ref[...] = jnp.dot(
        h, w_ref[...], preferred_element_type=jnp.float32,
        precision=lax.Precision.HIGHEST,
    ) * disb


_layer2 = pl.pallas_call(
    _layer2_body,
    grid=(GRID,),
    in_specs=[
        pl.BlockSpec((BLK, D), lambda i: (i, 0)),
        pl.BlockSpec((BLK, D), lambda i: (i + GRID, 0)),
        pl.BlockSpec((BLK, D), lambda i: (i, 0)),
        pl.BlockSpec((BLK, 16), lambda i: (i, 0)),
        pl.BlockSpec((1, D), lambda i: (0, 0)),
        pl.BlockSpec((D, D), lambda i: (0, 0)),
    ],
    out_specs=pl.BlockSpec((BLK, D), lambda i: (i, 0)),
    out_shape=jax.ShapeDtypeStruct((N, D), jnp.float32),
)


def _final_body(q0_ref, q1_ref, h2s_ref, dis_ref, bmu_ref, bls_ref, mu_ref, ls_ref):
    disb = dis_ref[:, 0:1]
    out = (q0_ref[...] + q1_ref[...] + h2s_ref[...]) * disb
    mu_ref[...] = out[:, :DOUT] + bmu_ref[...]
    ls_ref[...] = out[:, DOUT:] + bls_ref[...]


_final = pl.pallas_call(
    _final_body,
    grid=(GRID,),
    in_specs=[
        pl.BlockSpec((BLK, D), lambda i: (i, 0)),
        pl.BlockSpec((BLK, D), lambda i: (i + GRID, 0)),
        pl.BlockSpec((BLK, D), lambda i: (i, 0)),
        pl.BlockSpec((BLK, 16), lambda i: (i, 0)),
        pl.BlockSpec((1, DOUT), lambda i: (0, 0)),
        pl.BlockSpec((1, DOUT), lambda i: (0, 0)),
    ],
    out_specs=[
        pl.BlockSpec((BLK, DOUT), lambda i: (i, 0)),
        pl.BlockSpec((BLK, DOUT), lambda i: (i, 0)),
    ],
    out_shape=[
        jax.ShapeDtypeStruct((N, DOUT), jnp.float32),
        jax.ShapeDtypeStruct((N, DOUT), jnp.float32),
    ],
)


def kernel(x, edge_index, W1, b1, W_mu, b_mu, W_ls, b_ls):
    pad = E_PAD - E
    src_p = jnp.concatenate([edge_index[0], jnp.zeros((pad,), jnp.int32)])
    dst_p = jnp.concatenate([edge_index[1], jnp.full((pad,), TRASH, jnp.int32)])
    wcat = jnp.concatenate([W_mu, W_ls], axis=1)

    zerosD = jnp.zeros((N, D), jnp.float32)
    pd = _deg_kernel(dst_p, zerosD, jnp.ones((LANE, D), jnp.float32))
    h0s, dis = _layer1(x, W1, pd, pd)
    p = _prop_kernel(h0s, src_p, dst_p, zerosD)
    h2s = _layer2(p, p, h0s, dis, b1.reshape(1, D), wcat)
    q = _prop_kernel(h2s, src_p, dst_p, zerosD)
    mu, ls = _final(q, q, h2s, dis, b_mu.reshape(1, DOUT), b_ls.reshape(1, DOUT))
    return (mu, ls)


# speedup vs baseline: 11.1134x; 1.1033x over previous
import functools

import jax
import jax.numpy as jnp
from jax import lax
from jax.experimental import pallas as pl
from jax.experimental.pallas import tpu as pltpu
from jax.experimental.pallas import tpu_sc as plsc

N = 10000
E = 320000
D = 128
DOUT = 64
NC, NS = 2, 16
NW = NC * NS
LANE = 128
RW = 80
E_PAD = NW * RW * LANE
ROWS = E_PAD // LANE
TRASH = N
NACC = N + 8
RPA = 624
TAIL = N - NS * RPA
ZR = 104
R0, R1 = 120, 40

_mesh = plsc.VectorSubcoreMesh(
    core_axis_name="c", subcore_axis_name="s", num_cores=NC, num_subcores=NS
)


@functools.partial(
    pl.kernel,
    out_type=jax.ShapeDtypeStruct((2 * N, D), jnp.float32),
    mesh=_mesh,
    scratch_types=[
        pltpu.VMEM((LANE,), jnp.int32),
        pltpu.VMEM((LANE, D), jnp.float32),
        pltpu.VMEM_SHARED((NACC, D), jnp.float32),
    ],
)
def _deg_kernel(dst_hbm, z_hbm, ones_hbm, out_hbm, dstv, ones, acc):
    c = lax.axis_index("c")
    s = lax.axis_index("s")
    w = c * NS + s

    pltpu.sync_copy(ones_hbm, ones)

    pltpu.sync_copy(z_hbm.at[pl.ds(s * RPA, RPA)], acc.at[pl.ds(s * RPA, RPA)])

    @pl.when(s == NS - 1)
    def _():
        pltpu.sync_copy(
            z_hbm.at[pl.ds(NS * RPA, TAIL)], acc.at[pl.ds(NS * RPA, TAIL)]
        )

    plsc.subcore_barrier()

    @pl.loop(0, RW)
    def _(j):
        pltpu.sync_copy(dst_hbm.at[pl.ds((w * RW + j) * LANE, LANE)], dstv)
        pltpu.sync_copy(ones, acc.at[dstv], add=True)

    plsc.subcore_barrier()
    pltpu.sync_copy(
        acc.at[pl.ds(s * RPA, RPA)], out_hbm.at[pl.ds(c * N + s * RPA, RPA)]
    )

    @pl.when(s == NS - 1)
    def _():
        pltpu.sync_copy(
            acc.at[pl.ds(NS * RPA, TAIL)],
            out_hbm.at[pl.ds(c * N + NS * RPA, TAIL)],
        )


@functools.partial(
    pl.kernel,
    out_type=jax.ShapeDtypeStruct((2 * N, D), jnp.float32),
    mesh=_mesh,
    scratch_types=[
        pltpu.VMEM((LANE,), jnp.int32),
        pltpu.VMEM((LANE,), jnp.int32),
        pltpu.VMEM((LANE,), jnp.int32),
        pltpu.VMEM((LANE,), jnp.int32),
        pltpu.VMEM((LANE, D), jnp.float32),
        pltpu.VMEM((LANE, D), jnp.float32),
        pltpu.VMEM_SHARED((NACC, D), jnp.float32),
        pltpu.SemaphoreType.DMA,
        pltpu.SemaphoreType.DMA,
    ],
)
def _prop_kernel(h_hbm, src_hbm, dst_hbm, z_hbm, out_hbm,
                 srcv0, dstv0, srcv1, dstv1, buf0, buf1, acc, sg0, sg1):
    c = lax.axis_index("c")
    s = lax.axis_index("s")
    w = c * NS + s

    pltpu.sync_copy(z_hbm.at[pl.ds(s * RPA, RPA)], acc.at[pl.ds(s * RPA, RPA)])

    @pl.when(s == NS - 1)
    def _():
        pltpu.sync_copy(
            z_hbm.at[pl.ds(NS * RPA, TAIL)], acc.at[pl.ds(NS * RPA, TAIL)]
        )

    plsc.subcore_barrier()

    base = jnp.where(c == 0, s * R0, NS * R0 + s * R1)
    rw = jnp.where(c == 0, R0, R1)
    pltpu.sync_copy(src_hbm.at[pl.ds(base * LANE, LANE)], srcv0)
    pltpu.sync_copy(dst_hbm.at[pl.ds(base * LANE, LANE)], dstv0)
    pltpu.async_copy(h_hbm.at[srcv0], buf0, sg0)
    pltpu.sync_copy(src_hbm.at[pl.ds((base + 1) * LANE, LANE)], srcv1)
    pltpu.sync_copy(dst_hbm.at[pl.ds((base + 1) * LANE, LANE)], dstv1)

    @pl.loop(0, rw, step=2)
    def _(j):
        pltpu.make_async_copy(h_hbm.at[srcv0], buf0, sg0).wait()
        pltpu.async_copy(h_hbm.at[srcv1], buf1, sg1)
        pltpu.sync_copy(buf0, acc.at[dstv0], add=True)

        @pl.when(j + 2 < rw)
        def _():
            pltpu.sync_copy(
                src_hbm.at[pl.ds((base + j + 2) * LANE, LANE)], srcv0
            )
            pltpu.sync_copy(
                dst_hbm.at[pl.ds((base + j + 2) * LANE, LANE)], dstv0
            )

        pltpu.make_async_copy(h_hbm.at[srcv1], buf1, sg1).wait()

        @pl.when(j + 2 < rw)
        def _():
            pltpu.async_copy(h_hbm.at[srcv0], buf0, sg0)

        pltpu.sync_copy(buf1, acc.at[dstv1], add=True)

        @pl.when(j + 3 < rw)
        def _():
            pltpu.sync_copy(
                src_hbm.at[pl.ds((base + j + 3) * LANE, LANE)], srcv1
            )
            pltpu.sync_copy(
                dst_hbm.at[pl.ds((base + j + 3) * LANE, LANE)], dstv1
            )

    plsc.subcore_barrier()
    pltpu.sync_copy(
        acc.at[pl.ds(s * RPA, RPA)], out_hbm.at[pl.ds(c * N + s * RPA, RPA)]
    )

    @pl.when(s == NS - 1)
    def _():
        pltpu.sync_copy(
            acc.at[pl.ds(NS * RPA, TAIL)],
            out_hbm.at[pl.ds(c * N + NS * RPA, TAIL)],
        )


BLK = 1000
GRID = N // BLK


def _layer1_body(x_ref, w_ref, pd0_ref, pd1_ref, h0s_ref, dis_ref):
    deg = 1.0 + pd0_ref[:, 0:1] + pd1_ref[:, 0:1]
    dis = lax.rsqrt(deg)
    h0 = jnp.dot(
        x_ref[...], w_ref[...], preferred_element_type=jnp.float32,
        precision=lax.Precision.HIGHEST,
    )
    h0s_ref[...] = h0 * dis
    dis_ref[...] = jnp.broadcast_to(dis, (BLK, 16))


_layer1 = pl.pallas_call(
    _layer1_body,
    grid=(GRID,),
    in_specs=[
        pl.BlockSpec((BLK, D), lambda i: (i, 0)),
        pl.BlockSpec((D, D), lambda i: (0, 0)),
        pl.BlockSpec((BLK, D), lambda i: (i, 0)),
        pl.BlockSpec((BLK, D), lambda i: (i + GRID, 0)),
    ],
    out_specs=[
        pl.BlockSpec((BLK, D), lambda i: (i, 0)),
        pl.BlockSpec((BLK, 16), lambda i: (i, 0)),
    ],
    out_shape=[
        jax.ShapeDtypeStruct((N, D), jnp.float32),
        jax.ShapeDtypeStruct((N, 16), jnp.float32),
    ],
)


def _layer2_body(p0_ref, p1_ref, h0s_ref, dis_ref, b1_ref, w_ref, h2s_ref):
    disb = dis_ref[:, 0:1]
    h = jnp.maximum(
        (p0_ref[...] + p1_ref[...] + h0s_ref[...]) * disb + b1_ref[...], 0.0
    )
    h2s_ref[...] = jnp.dot(
        h, w_ref[...], preferred_element_type=jnp.float32,
        precision=lax.Precision.HIGHEST,
    ) * disb


_layer2 = pl.pallas_call(
    _layer2_body,
    grid=(GRID,),
    in_specs=[
        pl.BlockSpec((BLK, D), lambda i: (i, 0)),
        pl.BlockSpec((BLK, D), lambda i: (i + GRID, 0)),
        pl.BlockSpec((BLK, D), lambda i: (i, 0)),
        pl.BlockSpec((BLK, 16), lambda i: (i, 0)),
        pl.BlockSpec((1, D), lambda i: (0, 0)),
        pl.BlockSpec((D, D), lambda i: (0, 0)),
    ],
    out_specs=pl.BlockSpec((BLK, D), lambda i: (i, 0)),
    out_shape=jax.ShapeDtypeStruct((N, D), jnp.float32),
)


def _final_body(q0_ref, q1_ref, h2s_ref, dis_ref, bmu_ref, bls_ref, mu_ref, ls_ref):
    disb = dis_ref[:, 0:1]
    out = (q0_ref[...] + q1_ref[...] + h2s_ref[...]) * disb
    mu_ref[...] = out[:, :DOUT] + bmu_ref[...]
    ls_ref[...] = out[:, DOUT:] + bls_ref[...]


_final = pl.pallas_call(
    _final_body,
    grid=(GRID,),
    in_specs=[
        pl.BlockSpec((BLK, D), lambda i: (i, 0)),
        pl.BlockSpec((BLK, D), lambda i: (i + GRID, 0)),
        pl.BlockSpec((BLK, D), lambda i: (i, 0)),
        pl.BlockSpec((BLK, 16), lambda i: (i, 0)),
        pl.BlockSpec((1, DOUT), lambda i: (0, 0)),
        pl.BlockSpec((1, DOUT), lambda i: (0, 0)),
    ],
    out_specs=[
        pl.BlockSpec((BLK, DOUT), lambda i: (i, 0)),
        pl.BlockSpec((BLK, DOUT), lambda i: (i, 0)),
    ],
    out_shape=[
        jax.ShapeDtypeStruct((N, DOUT), jnp.float32),
        jax.ShapeDtypeStruct((N, DOUT), jnp.float32),
    ],
)


def kernel(x, edge_index, W1, b1, W_mu, b_mu, W_ls, b_ls):
    pad = E_PAD - E
    src_p = jnp.concatenate([edge_index[0], jnp.zeros((pad,), jnp.int32)])
    dst_p = jnp.concatenate([edge_index[1], jnp.full((pad,), TRASH, jnp.int32)])
    wcat = jnp.concatenate([W_mu, W_ls], axis=1)

    zerosD = jnp.zeros((N, D), jnp.float32)
    pd = _deg_kernel(dst_p, zerosD, jnp.ones((LANE, D), jnp.float32))
    h0s, dis = _layer1(x, W1, pd, pd)
    p = _prop_kernel(h0s, src_p, dst_p, zerosD)
    h2s = _layer2(p, p, h0s, dis, b1.reshape(1, D), wcat)
    q = _prop_kernel(h2s, src_p, dst_p, zerosD)
    mu, ls = _final(q, q, h2s, dis, b_mu.reshape(1, DOUT), b_ls.reshape(1, DOUT))
    return (mu, ls)
